# baseline (speedup 1.0000x reference)
import functools

import jax
import jax.numpy as jnp
from jax.experimental import pallas as pl
from jax.experimental.pallas import tpu as pltpu

_B = 8


def _body(f_ref, c_ref, out_ref, K: int, N: int):
    i = pl.program_id(0)
    f = f_ref[...]
    c = c_ref[...]
    B, D = f.shape

    sq = (f[:, None, :] - c[None, :, :]) ** 2
    min_idx = jnp.argmin(sq, axis=1).astype(jnp.int32)
    max_idx = jnp.argmax(sq, axis=1).astype(jnp.int32)

    def mode(idx):
        eq = (idx[:, :, None] == idx[:, None, :])
        cnt = jnp.sum(eq.astype(jnp.int32), axis=2)
        m = jnp.max(cnt, axis=1, keepdims=True)
        big = jnp.int32(0x7FFFFFFF)
        return jnp.min(jnp.where(cnt == m, idx, big), axis=1)

    mode_min = mode(min_idx)
    mode_max = mode(max_idx)

    karange = jax.lax.broadcasted_iota(jnp.int32, (B, K), 1)
    oh_min = (mode_min[:, None] == karange).astype(jnp.float32)
    oh_max = (mode_max[:, None] == karange).astype(jnp.float32)
    pos = jnp.dot(oh_min, c, preferred_element_type=jnp.float32)
    neg = jnp.dot(oh_max, c, preferred_element_type=jnp.float32)

    eps = jnp.float32(1e-6)

    def pdist(a, b):
        return jnp.sqrt(jnp.sum((a - b + eps) ** 2, axis=-1))

    d_ap = pdist(f, pos)
    d_an = pdist(f, neg)
    d_pn = pdist(pos, neg)
    d_neg = jnp.minimum(d_an, d_pn)
    part = jnp.sum(jnp.maximum(d_ap - d_neg + 1.0, 0.0))

    @pl.when(i == 0)
    def _():
        out_ref[0, 0] = jnp.float32(0.0)

    out_ref[0, 0] += part / jnp.float32(N)


def kernel(input_features, centroids):
    N, D = input_features.shape
    K, _ = centroids.shape

    out = pl.pallas_call(
        functools.partial(_body, K=K, N=N),
        grid=(N // _B,),
        in_specs=[
            pl.BlockSpec((_B, D), lambda i: (i, 0)),
            pl.BlockSpec((K, D), lambda i: (0, 0)),
        ],
        out_specs=pl.BlockSpec((1, 1), lambda i: (0, 0), memory_space=pltpu.SMEM),
        out_shape=jax.ShapeDtypeStruct((1, 1), jnp.float32),
    )(input_features, centroids)
    return out[0, 0]


# speedup vs baseline: 4.9822x; 4.9822x over previous
import functools

import jax
import jax.numpy as jnp
from jax import lax
from jax.experimental import pallas as pl
from jax.experimental.pallas import tpu as pltpu
from jax.experimental.pallas import tpu_sc as plsc

_INT_BIG = 0x7FFFFFFF


def _sort_body(cin_ref, sv_ref, ri_ref, ext_ref, K: int):
    v = cin_ref[...]
    D, KP = v.shape
    idx = lax.broadcasted_iota(jnp.int32, (D, KP), 1)
    lane = lax.broadcasted_iota(jnp.int32, (D, KP), 1)

    k = 2
    while k <= KP:
        j = k // 2
        while j >= 1:
            pv = jnp.where((lane & j) == 0,
                           jnp.roll(v, -j, axis=1), jnp.roll(v, j, axis=1))
            pi = jnp.where((lane & j) == 0,
                           jnp.roll(idx, -j, axis=1), jnp.roll(idx, j, axis=1))
            less = (v < pv) | ((v == pv) & (idx < pi))
            sel = ((lane & j) == 0) == ((lane & k) == 0)
            take = less == sel
            v = jnp.where(take, v, pv)
            idx = jnp.where(take, idx, pi)
            j //= 2
        k *= 2

    rm = idx
    s = 1
    while s < KP:
        sv_sh = jnp.roll(v, s, axis=1)
        rm_sh = jnp.roll(rm, s, axis=1)
        ok = (lane >= s) & (sv_sh == v)
        rm = jnp.where(ok, jnp.minimum(rm, rm_sh), rm)
        s *= 2

    sv_ref[...] = v
    ri_ref[...] = rm

    ones = jnp.ones((1, 16), jnp.float32)
    vmin16 = v[:, 0:1] * ones
    vmax16 = v[:, K - 1:K] * ones
    iones = jnp.ones((1, 16), jnp.int32)
    imin16 = lax.bitcast_convert_type(rm[:, 0:1] * iones, jnp.float32)
    imax16 = lax.bitcast_convert_type(rm[:, K - 1:K] * iones, jnp.float32)
    ext_ref[...] = jnp.concatenate([vmin16, vmax16, imin16, imax16], axis=1)


def _sort_columns(cT_inf, K):
    D, KP = cT_inf.shape
    return pl.pallas_call(
        functools.partial(_sort_body, K=K),
        out_shape=[
            jax.ShapeDtypeStruct((D, KP), jnp.float32),
            jax.ShapeDtypeStruct((D, KP), jnp.int32),
            jax.ShapeDtypeStruct((D, 64), jnp.float32),
        ],
    )(cT_inf)


def _search_tc_rows(qrow, svrow, rirow, extrow, omrow, oxrow, N, K):
    KP = 1024
    inf = jnp.float32(jnp.inf)

    def batch(i, _):
        vmin = extrow[pl.ds(0, 16)]
        vmax = extrow[pl.ds(16, 16)]
        imin = plsc.bitcast(extrow[pl.ds(32, 16)], jnp.int32)
        imax = plsc.bitcast(extrow[pl.ds(48, 16)], jnp.int32)
        q = qrow[pl.ds(i * 16, 16)]
        p = jnp.zeros((16,), jnp.int32)
        s = KP // 2
        while s >= 1:
            probe = p + (s - 1)
            val = plsc.load_gather(svrow, [probe])
            p = p + jnp.where(val < q, s, 0)
            s //= 2
        pred = jnp.maximum(p - 1, 0)
        v_pred = plsc.load_gather(svrow, [pred])
        i_pred = plsc.load_gather(rirow, [pred])
        v_succ = plsc.load_gather(svrow, [p])
        i_succ = plsc.load_gather(rirow, [p])
        sq_pred = jnp.where(p == 0, inf, (q - v_pred) * (q - v_pred))
        sq_succ = (q - v_succ) * (q - v_succ)
        minidx = jnp.where(
            sq_pred < sq_succ, i_pred,
            jnp.where(sq_succ < sq_pred, i_succ, jnp.minimum(i_pred, i_succ)))
        sqmin = (q - vmin) * (q - vmin)
        sqmax = (q - vmax) * (q - vmax)
        maxidx = jnp.where(
            sqmin > sqmax, imin,
            jnp.where(sqmax > sqmin, imax, jnp.minimum(imin, imax)))
        omrow[pl.ds(i * 16, 16)] = minidx
        oxrow[pl.ds(i * 16, 16)] = maxidx
        return 0

    lax.fori_loop(0, N // 16, batch, 0)


def _sc_search(fT, svals, ridx, ext, K):
    D, N = fT.shape
    KP = svals.shape[1]
    mesh = plsc.VectorSubcoreMesh(core_axis_name="c", subcore_axis_name="s")
    nw = mesh.num_cores * mesh.num_subcores
    rows_per_w = D // nw

    per_row = [
        pltpu.VMEM((N,), jnp.float32),
        pltpu.VMEM((KP,), jnp.float32),
        pltpu.VMEM((KP,), jnp.int32),
        pltpu.VMEM((64,), jnp.float32),
        pltpu.VMEM((N,), jnp.int32),
        pltpu.VMEM((N,), jnp.int32),
    ]

    @functools.partial(
        pl.kernel,
        out_type=[
            jax.ShapeDtypeStruct((D, N), jnp.int32),
            jax.ShapeDtypeStruct((D, N), jnp.int32),
        ],
        mesh=mesh,
        compiler_params=pltpu.CompilerParams(needs_layout_passes=False),
        scratch_types=per_row * rows_per_w,
    )
    def k(fT_hbm, sv_hbm, ri_hbm, ext_hbm, omin_hbm, omax_hbm, *scratch):
        wid = lax.axis_index("s") * mesh.num_cores + lax.axis_index("c")
        for r in range(rows_per_w):
            q_v, sv_v, ri_v, ext_v, om_v, ox_v = scratch[6 * r:6 * r + 6]
            row = wid * rows_per_w + r
            pltpu.sync_copy(fT_hbm.at[row], q_v)
            pltpu.sync_copy(sv_hbm.at[row], sv_v)
            pltpu.sync_copy(ri_hbm.at[row], ri_v)
            pltpu.sync_copy(ext_hbm.at[row], ext_v)
            _search_tc_rows(q_v, sv_v, ri_v, ext_v, om_v, ox_v, N, K)
            pltpu.sync_copy(om_v, omin_hbm.at[row])
            pltpu.sync_copy(ox_v, omax_hbm.at[row])

    return k(fT, svals, ridx, ext)


def _loss_body(fT_ref, cT0_ref, minT_ref, maxT_ref, out_ref):
    fT = fT_ref[...]
    cT0 = cT0_ref[...]
    D, N = fT.shape

    def mode_of(idxT):
        cnt = jnp.zeros(idxT.shape, jnp.int32)
        for e in range(D):
            cnt = cnt + (idxT == idxT[e:e + 1, :]).astype(jnp.int32)
        m = jnp.max(cnt, axis=0, keepdims=True)
        return jnp.min(jnp.where(cnt == m, idxT, _INT_BIG), axis=0,
                       keepdims=True)

    mode_min = mode_of(minT_ref[...])
    mode_max = mode_of(maxT_ref[...])

    KP = cT0.shape[1]
    krange = lax.broadcasted_iota(jnp.int32, (KP, N), 0)
    oh_min = (krange == mode_min).astype(jnp.float32)
    oh_max = (krange == mode_max).astype(jnp.float32)
    posT = jnp.dot(cT0, oh_min, preferred_element_type=jnp.float32)
    negT = jnp.dot(cT0, oh_max, preferred_element_type=jnp.float32)

    eps = jnp.float32(1e-6)

    def pdist(a, b):
        return jnp.sqrt(jnp.sum((a - b + eps) ** 2, axis=0, keepdims=True))

    d_ap = pdist(fT, posT)
    d_an = pdist(fT, negT)
    d_pn = pdist(posT, negT)
    d_neg = jnp.minimum(d_an, d_pn)
    loss = jnp.sum(jnp.maximum(d_ap - d_neg + 1.0, 0.0)) / jnp.float32(N)
    out_ref[0, 0] = loss


def _mode_and_loss(fT, cT0, minT, maxT):
    out = pl.pallas_call(
        _loss_body,
        out_specs=pl.BlockSpec(memory_space=pltpu.SMEM),
        out_shape=jax.ShapeDtypeStruct((1, 1), jnp.float32),
    )(fT, cT0, minT, maxT)
    return out[0, 0]


def kernel(input_features, centroids):
    N, D = input_features.shape
    K, _ = centroids.shape
    KP = 1024
    assert K <= KP

    fT = input_features.T
    cT = centroids.T
    pad_inf = jnp.full((D, KP - K), jnp.inf, jnp.float32)
    pad_zero = jnp.zeros((D, KP - K), jnp.float32)
    cT_inf = jnp.concatenate([cT, pad_inf], axis=1)
    cT0 = jnp.concatenate([cT, pad_zero], axis=1)

    svals, ridx, ext = _sort_columns(cT_inf, K)
    minT, maxT = _sc_search(fT, svals, ridx, ext, K)
    return _mode_and_loss(fT, cT0, minT, maxT)


# speedup vs baseline: 5.7237x; 1.1488x over previous
import functools

import jax
import jax.numpy as jnp
from jax import lax
from jax.experimental import pallas as pl
from jax.experimental.pallas import tpu as pltpu
from jax.experimental.pallas import tpu_sc as plsc

_INT_BIG = 0x7FFFFFFF


def _sort_body(cin_ref, sv_ref, ri_ref, ext_ref, K: int):
    v = cin_ref[...]
    D, KP = v.shape
    idx = lax.broadcasted_iota(jnp.int32, (D, KP), 1)
    lane = lax.broadcasted_iota(jnp.int32, (D, KP), 1)

    k = 2
    while k <= KP:
        j = k // 2
        while j >= 1:
            pv = jnp.where((lane & j) == 0,
                           jnp.roll(v, -j, axis=1), jnp.roll(v, j, axis=1))
            pi = jnp.where((lane & j) == 0,
                           jnp.roll(idx, -j, axis=1), jnp.roll(idx, j, axis=1))
            less = (v < pv) | ((v == pv) & (idx < pi))
            sel = ((lane & j) == 0) == ((lane & k) == 0)
            take = less == sel
            v = jnp.where(take, v, pv)
            idx = jnp.where(take, idx, pi)
            j //= 2
        k *= 2

    rm = idx
    s = 1
    while s < KP:
        sv_sh = jnp.roll(v, s, axis=1)
        rm_sh = jnp.roll(rm, s, axis=1)
        ok = (lane >= s) & (sv_sh == v)
        rm = jnp.where(ok, jnp.minimum(rm, rm_sh), rm)
        s *= 2

    sv_ref[...] = v
    ri_ref[...] = rm

    ones = jnp.ones((1, 16), jnp.float32)
    vmin16 = v[:, 0:1] * ones
    vmax16 = v[:, K - 1:K] * ones
    iones = jnp.ones((1, 16), jnp.int32)
    imin16 = lax.bitcast_convert_type(rm[:, 0:1] * iones, jnp.float32)
    imax16 = lax.bitcast_convert_type(rm[:, K - 1:K] * iones, jnp.float32)
    ext_ref[...] = jnp.concatenate([vmin16, vmax16, imin16, imax16], axis=1)


def _sort_columns(cT_inf, K):
    D, KP = cT_inf.shape
    return pl.pallas_call(
        functools.partial(_sort_body, K=K),
        out_shape=[
            jax.ShapeDtypeStruct((D, KP), jnp.float32),
            jax.ShapeDtypeStruct((D, KP), jnp.int32),
            jax.ShapeDtypeStruct((D, 64), jnp.float32),
        ],
    )(cT_inf)


def _search_tc_rows(qrow, svrow, rirow, extrow, omrow, oxrow, N, K):
    KP = 1024
    inf = jnp.float32(jnp.inf)
    NCH = 2

    def batch(i, _):
        vmin = extrow[pl.ds(0, 16)]
        vmax = extrow[pl.ds(16, 16)]
        imin = plsc.bitcast(extrow[pl.ds(32, 16)], jnp.int32)
        imax = plsc.bitcast(extrow[pl.ds(48, 16)], jnp.int32)
        base = i * (16 * NCH)
        qs = [qrow[pl.ds(base + 16 * c, 16)] for c in range(NCH)]
        ps = [jnp.zeros((16,), jnp.int32) for _ in range(NCH)]
        s = KP // 2
        while s >= 1:
            for c in range(NCH):
                probe = ps[c] + (s - 1)
                val = plsc.load_gather(svrow, [probe])
                ps[c] = ps[c] + jnp.where(val < qs[c], s, 0)
            s //= 2
        for c in range(NCH):
            q, p = qs[c], ps[c]
            pred = jnp.maximum(p - 1, 0)
            v_pred = plsc.load_gather(svrow, [pred])
            i_pred = plsc.load_gather(rirow, [pred])
            v_succ = plsc.load_gather(svrow, [p])
            i_succ = plsc.load_gather(rirow, [p])
            sq_pred = jnp.where(p == 0, inf, (q - v_pred) * (q - v_pred))
            sq_succ = (q - v_succ) * (q - v_succ)
            minidx = jnp.where(
                sq_pred < sq_succ, i_pred,
                jnp.where(sq_succ < sq_pred, i_succ,
                          jnp.minimum(i_pred, i_succ)))
            sqmin = (q - vmin) * (q - vmin)
            sqmax = (q - vmax) * (q - vmax)
            maxidx = jnp.where(
                sqmin > sqmax, imin,
                jnp.where(sqmax > sqmin, imax, jnp.minimum(imin, imax)))
            omrow[pl.ds(base + 16 * c, 16)] = minidx
            oxrow[pl.ds(base + 16 * c, 16)] = maxidx
        return 0

    lax.fori_loop(0, N // (16 * NCH), batch, 0)


def _sc_search(fT, svals, ridx, ext, K):
    D, N = fT.shape
    KP = svals.shape[1]
    mesh = plsc.VectorSubcoreMesh(core_axis_name="c", subcore_axis_name="s")
    nw = mesh.num_cores * mesh.num_subcores
    rows_per_w = D // nw

    per_row = [
        pltpu.VMEM((N,), jnp.float32),
        pltpu.VMEM((KP,), jnp.float32),
        pltpu.VMEM((KP,), jnp.int32),
        pltpu.VMEM((64,), jnp.float32),
        pltpu.VMEM((N,), jnp.int32),
        pltpu.VMEM((N,), jnp.int32),
    ]

    @functools.partial(
        pl.kernel,
        out_type=[
            jax.ShapeDtypeStruct((D, N), jnp.int32),
            jax.ShapeDtypeStruct((D, N), jnp.int32),
        ],
        mesh=mesh,
        compiler_params=pltpu.CompilerParams(needs_layout_passes=False),
        scratch_types=per_row * rows_per_w
        + [pltpu.SemaphoreType.DMA] * rows_per_w
        + [pltpu.SemaphoreType.DMA],
    )
    def k(fT_hbm, sv_hbm, ri_hbm, ext_hbm, omin_hbm, omax_hbm, *scratch):
        wid = lax.axis_index("s") * mesh.num_cores + lax.axis_index("c")
        sems = scratch[6 * rows_per_w:6 * rows_per_w + rows_per_w]
        osem = scratch[6 * rows_per_w + rows_per_w]
        handles = []
        for r in range(rows_per_w):
            q_v, sv_v, ri_v, ext_v, _, _ = scratch[6 * r:6 * r + 6]
            row = wid * rows_per_w + r
            handles.append([
                pltpu.async_copy(fT_hbm.at[row], q_v, sems[r]),
                pltpu.async_copy(sv_hbm.at[row], sv_v, sems[r]),
                pltpu.async_copy(ri_hbm.at[row], ri_v, sems[r]),
                pltpu.async_copy(ext_hbm.at[row], ext_v, sems[r]),
            ])
        out_handles = []
        for r in range(rows_per_w):
            q_v, sv_v, ri_v, ext_v, om_v, ox_v = scratch[6 * r:6 * r + 6]
            row = wid * rows_per_w + r
            for h in handles[r]:
                h.wait()
            _search_tc_rows(q_v, sv_v, ri_v, ext_v, om_v, ox_v, N, K)
            out_handles.append(pltpu.async_copy(om_v, omin_hbm.at[row], osem))
            out_handles.append(pltpu.async_copy(ox_v, omax_hbm.at[row], osem))
        for h in out_handles:
            h.wait()

    return k(fT, svals, ridx, ext)


def _loss_body(fT_ref, cT0_ref, minT_ref, maxT_ref, out_ref):
    fT = fT_ref[...]
    cT0 = cT0_ref[...]
    D, N = fT.shape

    def mode_of(idxT):
        cnt = jnp.zeros(idxT.shape, jnp.int32)
        for e in range(D):
            cnt = cnt + (idxT == idxT[e:e + 1, :]).astype(jnp.int32)
        m = jnp.max(cnt, axis=0, keepdims=True)
        return jnp.min(jnp.where(cnt == m, idxT, _INT_BIG), axis=0,
                       keepdims=True)

    mode_min = mode_of(minT_ref[...])
    mode_max = mode_of(maxT_ref[...])

    KP = cT0.shape[1]
    krange = lax.broadcasted_iota(jnp.int32, (KP, N), 0)
    oh_min = (krange == mode_min).astype(jnp.float32)
    oh_max = (krange == mode_max).astype(jnp.float32)
    posT = jnp.dot(cT0, oh_min, preferred_element_type=jnp.float32)
    negT = jnp.dot(cT0, oh_max, preferred_element_type=jnp.float32)

    eps = jnp.float32(1e-6)

    def pdist(a, b):
        return jnp.sqrt(jnp.sum((a - b + eps) ** 2, axis=0, keepdims=True))

    d_ap = pdist(fT, posT)
    d_an = pdist(fT, negT)
    d_pn = pdist(posT, negT)
    d_neg = jnp.minimum(d_an, d_pn)
    loss = jnp.sum(jnp.maximum(d_ap - d_neg + 1.0, 0.0)) / jnp.float32(N)
    out_ref[0, 0] = loss


def _mode_and_loss(fT, cT0, minT, maxT):
    out = pl.pallas_call(
        _loss_body,
        out_specs=pl.BlockSpec(memory_space=pltpu.SMEM),
        out_shape=jax.ShapeDtypeStruct((1, 1), jnp.float32),
    )(fT, cT0, minT, maxT)
    return out[0, 0]


def kernel(input_features, centroids):
    N, D = input_features.shape
    K, _ = centroids.shape
    KP = 1024
    assert K <= KP

    fT = input_features.T
    cT = centroids.T
    pad_inf = jnp.full((D, KP - K), jnp.inf, jnp.float32)
    pad_zero = jnp.zeros((D, KP - K), jnp.float32)
    cT_inf = jnp.concatenate([cT, pad_inf], axis=1)
    cT0 = jnp.concatenate([cT, pad_zero], axis=1)

    svals, ridx, ext = _sort_columns(cT_inf, K)
    minT, maxT = _sc_search(fT, svals, ridx, ext, K)
    return _mode_and_loss(fT, cT0, minT, maxT)


# speedup vs baseline: 5.9824x; 1.0452x over previous
import functools

import jax
import jax.numpy as jnp
from jax import lax
from jax.experimental import pallas as pl
from jax.experimental.pallas import tpu as pltpu
from jax.experimental.pallas import tpu_sc as plsc

_INT_BIG = 0x7FFFFFFF


def _sort_body(cin_ref, sv_ref, ri_ref, ext_ref, K: int):
    v = cin_ref[...]
    D, KP = v.shape
    idx = lax.broadcasted_iota(jnp.int32, (D, KP), 1)
    lane = lax.broadcasted_iota(jnp.int32, (D, KP), 1)

    k = 2
    while k <= KP:
        j = k // 2
        while j >= 1:
            pv = jnp.where((lane & j) == 0,
                           jnp.roll(v, -j, axis=1), jnp.roll(v, j, axis=1))
            pi = jnp.where((lane & j) == 0,
                           jnp.roll(idx, -j, axis=1), jnp.roll(idx, j, axis=1))
            less = (v < pv) | ((v == pv) & (idx < pi))
            sel = ((lane & j) == 0) == ((lane & k) == 0)
            take = less == sel
            v = jnp.where(take, v, pv)
            idx = jnp.where(take, idx, pi)
            j //= 2
        k *= 2

    rm = idx
    s = 1
    while s < KP:
        sv_sh = jnp.roll(v, s, axis=1)
        rm_sh = jnp.roll(rm, s, axis=1)
        ok = (lane >= s) & (sv_sh == v)
        rm = jnp.where(ok, jnp.minimum(rm, rm_sh), rm)
        s *= 2

    sv_ref[...] = v
    ri_ref[...] = rm

    ones = jnp.ones((1, 16), jnp.float32)
    vmin16 = v[:, 0:1] * ones
    vmax16 = v[:, K - 1:K] * ones
    iones = jnp.ones((1, 16), jnp.int32)
    imin16 = lax.bitcast_convert_type(rm[:, 0:1] * iones, jnp.float32)
    imax16 = lax.bitcast_convert_type(rm[:, K - 1:K] * iones, jnp.float32)
    ext_ref[...] = jnp.concatenate([vmin16, vmax16, imin16, imax16], axis=1)


def _sort_columns(cT_inf, K):
    D, KP = cT_inf.shape
    return pl.pallas_call(
        functools.partial(_sort_body, K=K),
        out_shape=[
            jax.ShapeDtypeStruct((D, KP), jnp.float32),
            jax.ShapeDtypeStruct((D, KP), jnp.int32),
            jax.ShapeDtypeStruct((D, 64), jnp.float32),
        ],
    )(cT_inf)


def _search_tc_rows(qrow, svrow, rirow, extrow, omrow, oxrow, N, K):
    KP = 1024
    inf = jnp.float32(jnp.inf)
    NCH = 4

    def batch(i, _):
        vmin = extrow[pl.ds(0, 16)]
        vmax = extrow[pl.ds(16, 16)]
        imin = plsc.bitcast(extrow[pl.ds(32, 16)], jnp.int32)
        imax = plsc.bitcast(extrow[pl.ds(48, 16)], jnp.int32)
        base = i * (16 * NCH)
        qs = [qrow[pl.ds(base + 16 * c, 16)] for c in range(NCH)]
        ps = [jnp.zeros((16,), jnp.int32) for _ in range(NCH)]
        s = KP // 2
        while s >= 1:
            for c in range(NCH):
                probe = ps[c] + (s - 1)
                val = plsc.load_gather(svrow, [probe])
                ps[c] = ps[c] + jnp.where(val < qs[c], s, 0)
            s //= 2
        for c in range(NCH):
            q, p = qs[c], ps[c]
            pred = jnp.maximum(p - 1, 0)
            v_pred = plsc.load_gather(svrow, [pred])
            i_pred = plsc.load_gather(rirow, [pred])
            v_succ = plsc.load_gather(svrow, [p])
            i_succ = plsc.load_gather(rirow, [p])
            sq_pred = jnp.where(p == 0, inf, (q - v_pred) * (q - v_pred))
            sq_succ = (q - v_succ) * (q - v_succ)
            minidx = jnp.where(
                sq_pred < sq_succ, i_pred,
                jnp.where(sq_succ < sq_pred, i_succ,
                          jnp.minimum(i_pred, i_succ)))
            sqmin = (q - vmin) * (q - vmin)
            sqmax = (q - vmax) * (q - vmax)
            maxidx = jnp.where(
                sqmin > sqmax, imin,
                jnp.where(sqmax > sqmin, imax, jnp.minimum(imin, imax)))
            omrow[pl.ds(base + 16 * c, 16)] = minidx
            oxrow[pl.ds(base + 16 * c, 16)] = maxidx
        return 0

    lax.fori_loop(0, N // (16 * NCH), batch, 0)


def _sc_search(fT, svals, ridx, ext, K):
    D, N = fT.shape
    KP = svals.shape[1]
    mesh = plsc.VectorSubcoreMesh(core_axis_name="c", subcore_axis_name="s")
    nw = mesh.num_cores * mesh.num_subcores
    rows_per_w = D // nw

    per_row = [
        pltpu.VMEM((N,), jnp.float32),
        pltpu.VMEM((KP,), jnp.float32),
        pltpu.VMEM((KP,), jnp.int32),
        pltpu.VMEM((64,), jnp.float32),
        pltpu.VMEM((N,), jnp.int32),
        pltpu.VMEM((N,), jnp.int32),
    ]

    @functools.partial(
        pl.kernel,
        out_type=[
            jax.ShapeDtypeStruct((D, N), jnp.int32),
            jax.ShapeDtypeStruct((D, N), jnp.int32),
        ],
        mesh=mesh,
        compiler_params=pltpu.CompilerParams(needs_layout_passes=False),
        scratch_types=per_row * rows_per_w
        + [pltpu.SemaphoreType.DMA] * rows_per_w
        + [pltpu.SemaphoreType.DMA],
    )
    def k(fT_hbm, sv_hbm, ri_hbm, ext_hbm, omin_hbm, omax_hbm, *scratch):
        wid = lax.axis_index("s") * mesh.num_cores + lax.axis_index("c")
        sems = scratch[6 * rows_per_w:6 * rows_per_w + rows_per_w]
        osem = scratch[6 * rows_per_w + rows_per_w]
        handles = []
        for r in range(rows_per_w):
            q_v, sv_v, ri_v, ext_v, _, _ = scratch[6 * r:6 * r + 6]
            row = wid * rows_per_w + r
            handles.append([
                pltpu.async_copy(fT_hbm.at[row], q_v, sems[r]),
                pltpu.async_copy(sv_hbm.at[row], sv_v, sems[r]),
                pltpu.async_copy(ri_hbm.at[row], ri_v, sems[r]),
                pltpu.async_copy(ext_hbm.at[row], ext_v, sems[r]),
            ])
        out_handles = []
        for r in range(rows_per_w):
            q_v, sv_v, ri_v, ext_v, om_v, ox_v = scratch[6 * r:6 * r + 6]
            row = wid * rows_per_w + r
            for h in handles[r]:
                h.wait()
            _search_tc_rows(q_v, sv_v, ri_v, ext_v, om_v, ox_v, N, K)
            out_handles.append(pltpu.async_copy(om_v, omin_hbm.at[row], osem))
            out_handles.append(pltpu.async_copy(ox_v, omax_hbm.at[row], osem))
        for h in out_handles:
            h.wait()

    return k(fT, svals, ridx, ext)


def _loss_body(fT_ref, cT0_ref, minT_ref, maxT_ref, out_ref):
    fT = fT_ref[...]
    cT0 = cT0_ref[...]
    D, N = fT.shape

    def mode_of(idxT):
        cnt = jnp.zeros(idxT.shape, jnp.int32)
        for e in range(D):
            cnt = cnt + (idxT == idxT[e:e + 1, :]).astype(jnp.int32)
        m = jnp.max(cnt, axis=0, keepdims=True)
        return jnp.min(jnp.where(cnt == m, idxT, _INT_BIG), axis=0,
                       keepdims=True)

    mode_min = mode_of(minT_ref[...])
    mode_max = mode_of(maxT_ref[...])

    KP = cT0.shape[1]
    krange = lax.broadcasted_iota(jnp.int32, (KP, N), 0)
    oh_min = (krange == mode_min).astype(jnp.float32)
    oh_max = (krange == mode_max).astype(jnp.float32)
    posT = jnp.dot(cT0, oh_min, preferred_element_type=jnp.float32)
    negT = jnp.dot(cT0, oh_max, preferred_element_type=jnp.float32)

    eps = jnp.float32(1e-6)

    def pdist(a, b):
        return jnp.sqrt(jnp.sum((a - b + eps) ** 2, axis=0, keepdims=True))

    d_ap = pdist(fT, posT)
    d_an = pdist(fT, negT)
    d_pn = pdist(posT, negT)
    d_neg = jnp.minimum(d_an, d_pn)
    loss = jnp.sum(jnp.maximum(d_ap - d_neg + 1.0, 0.0)) / jnp.float32(N)
    out_ref[0, 0] = loss


def _mode_and_loss(fT, cT0, minT, maxT):
    out = pl.pallas_call(
        _loss_body,
        out_specs=pl.BlockSpec(memory_space=pltpu.SMEM),
        out_shape=jax.ShapeDtypeStruct((1, 1), jnp.float32),
    )(fT, cT0, minT, maxT)
    return out[0, 0]


def kernel(input_features, centroids):
    N, D = input_features.shape
    K, _ = centroids.shape
    KP = 1024
    assert K <= KP

    fT = input_features.T
    cT = centroids.T
    pad_inf = jnp.full((D, KP - K), jnp.inf, jnp.float32)
    pad_zero = jnp.zeros((D, KP - K), jnp.float32)
    cT_inf = jnp.concatenate([cT, pad_inf], axis=1)
    cT0 = jnp.concatenate([cT, pad_zero], axis=1)

    svals, ridx, ext = _sort_columns(cT_inf, K)
    minT, maxT = _sc_search(fT, svals, ridx, ext, K)
    return _mode_and_loss(fT, cT0, minT, maxT)


# speedup vs baseline: 6.1062x; 1.0207x over previous
import functools

import jax
import jax.numpy as jnp
from jax import lax
from jax.experimental import pallas as pl
from jax.experimental.pallas import tpu as pltpu
from jax.experimental.pallas import tpu_sc as plsc

_INT_BIG = 0x7FFFFFFF


def _sort_body(cin_ref, sv_ref, ri_ref, ext_ref, K: int):
    v = cin_ref[...]
    D, KP = v.shape
    idx = lax.broadcasted_iota(jnp.int32, (D, KP), 1)
    lane = lax.broadcasted_iota(jnp.int32, (D, KP), 1)

    k = 2
    while k <= KP:
        j = k // 2
        while j >= 1:
            pv = jnp.where((lane & j) == 0,
                           jnp.roll(v, -j, axis=1), jnp.roll(v, j, axis=1))
            pi = jnp.where((lane & j) == 0,
                           jnp.roll(idx, -j, axis=1), jnp.roll(idx, j, axis=1))
            less = (v < pv) | ((v == pv) & (idx < pi))
            sel = ((lane & j) == 0) == ((lane & k) == 0)
            take = less == sel
            v = jnp.where(take, v, pv)
            idx = jnp.where(take, idx, pi)
            j //= 2
        k *= 2

    rm = idx
    s = 1
    while s < KP:
        sv_sh = jnp.roll(v, s, axis=1)
        rm_sh = jnp.roll(rm, s, axis=1)
        ok = (lane >= s) & (sv_sh == v)
        rm = jnp.where(ok, jnp.minimum(rm, rm_sh), rm)
        s *= 2

    sv_ref[...] = v
    ri_ref[...] = rm

    ones = jnp.ones((1, 16), jnp.float32)
    vmin16 = v[:, 0:1] * ones
    vmax16 = v[:, K - 1:K] * ones
    iones = jnp.ones((1, 16), jnp.int32)
    imin16 = lax.bitcast_convert_type(rm[:, 0:1] * iones, jnp.float32)
    imax16 = lax.bitcast_convert_type(rm[:, K - 1:K] * iones, jnp.float32)
    ext_ref[...] = jnp.concatenate([vmin16, vmax16, imin16, imax16], axis=1)


def _sort_columns(cT_inf, K):
    D, KP = cT_inf.shape
    return pl.pallas_call(
        functools.partial(_sort_body, K=K),
        out_shape=[
            jax.ShapeDtypeStruct((D, KP), jnp.float32),
            jax.ShapeDtypeStruct((D, KP), jnp.int32),
            jax.ShapeDtypeStruct((D, 64), jnp.float32),
        ],
    )(cT_inf)


def _search_tc_rows(qrow, svrow, rirow, extrow, omrow, oxrow, N, K):
    KP = 1024
    inf = jnp.float32(jnp.inf)
    NCH = 8

    def batch(i, _):
        vmin = extrow[pl.ds(0, 16)]
        vmax = extrow[pl.ds(16, 16)]
        imin = plsc.bitcast(extrow[pl.ds(32, 16)], jnp.int32)
        imax = plsc.bitcast(extrow[pl.ds(48, 16)], jnp.int32)
        base = i * (16 * NCH)
        qs = [qrow[pl.ds(base + 16 * c, 16)] for c in range(NCH)]
        ps = [jnp.zeros((16,), jnp.int32) for _ in range(NCH)]
        s = KP // 2
        while s >= 1:
            for c in range(NCH):
                probe = ps[c] + (s - 1)
                val = plsc.load_gather(svrow, [probe])
                ps[c] = ps[c] + jnp.where(val < qs[c], s, 0)
            s //= 2
        for c in range(NCH):
            q, p = qs[c], ps[c]
            pred = jnp.maximum(p - 1, 0)
            v_pred = plsc.load_gather(svrow, [pred])
            i_pred = plsc.load_gather(rirow, [pred])
            v_succ = plsc.load_gather(svrow, [p])
            i_succ = plsc.load_gather(rirow, [p])
            sq_pred = jnp.where(p == 0, inf, (q - v_pred) * (q - v_pred))
            sq_succ = (q - v_succ) * (q - v_succ)
            minidx = jnp.where(
                sq_pred < sq_succ, i_pred,
                jnp.where(sq_succ < sq_pred, i_succ,
                          jnp.minimum(i_pred, i_succ)))
            sqmin = (q - vmin) * (q - vmin)
            sqmax = (q - vmax) * (q - vmax)
            maxidx = jnp.where(
                sqmin > sqmax, imin,
                jnp.where(sqmax > sqmin, imax, jnp.minimum(imin, imax)))
            omrow[pl.ds(base + 16 * c, 16)] = minidx
            oxrow[pl.ds(base + 16 * c, 16)] = maxidx
        return 0

    lax.fori_loop(0, N // (16 * NCH), batch, 0)


def _sc_search(fT, svals, ridx, ext, K):
    D, N = fT.shape
    KP = svals.shape[1]
    mesh = plsc.VectorSubcoreMesh(core_axis_name="c", subcore_axis_name="s")
    nw = mesh.num_cores * mesh.num_subcores
    rows_per_w = D // nw

    per_row = [
        pltpu.VMEM((N,), jnp.float32),
        pltpu.VMEM((KP,), jnp.float32),
        pltpu.VMEM((KP,), jnp.int32),
        pltpu.VMEM((64,), jnp.float32),
        pltpu.VMEM((N,), jnp.int32),
        pltpu.VMEM((N,), jnp.int32),
    ]

    @functools.partial(
        pl.kernel,
        out_type=[
            jax.ShapeDtypeStruct((D, N), jnp.int32),
            jax.ShapeDtypeStruct((D, N), jnp.int32),
        ],
        mesh=mesh,
        compiler_params=pltpu.CompilerParams(needs_layout_passes=False),
        scratch_types=per_row * rows_per_w
        + [pltpu.SemaphoreType.DMA] * rows_per_w
        + [pltpu.SemaphoreType.DMA],
    )
    def k(fT_hbm, sv_hbm, ri_hbm, ext_hbm, omin_hbm, omax_hbm, *scratch):
        wid = lax.axis_index("s") * mesh.num_cores + lax.axis_index("c")
        sems = scratch[6 * rows_per_w:6 * rows_per_w + rows_per_w]
        osem = scratch[6 * rows_per_w + rows_per_w]
        handles = []
        for r in range(rows_per_w):
            q_v, sv_v, ri_v, ext_v, _, _ = scratch[6 * r:6 * r + 6]
            row = wid * rows_per_w + r
            handles.append([
                pltpu.async_copy(fT_hbm.at[row], q_v, sems[r]),
                pltpu.async_copy(sv_hbm.at[row], sv_v, sems[r]),
                pltpu.async_copy(ri_hbm.at[row], ri_v, sems[r]),
                pltpu.async_copy(ext_hbm.at[row], ext_v, sems[r]),
            ])
        out_handles = []
        for r in range(rows_per_w):
            q_v, sv_v, ri_v, ext_v, om_v, ox_v = scratch[6 * r:6 * r + 6]
            row = wid * rows_per_w + r
            for h in handles[r]:
                h.wait()
            _search_tc_rows(q_v, sv_v, ri_v, ext_v, om_v, ox_v, N, K)
            out_handles.append(pltpu.async_copy(om_v, omin_hbm.at[row], osem))
            out_handles.append(pltpu.async_copy(ox_v, omax_hbm.at[row], osem))
        for h in out_handles:
            h.wait()

    return k(fT, svals, ridx, ext)


def _loss_body(fT_ref, cT0_ref, minT_ref, maxT_ref, out_ref):
    fT = fT_ref[...]
    cT0 = cT0_ref[...]
    D, N = fT.shape

    def mode_of(idxT):
        cnt = jnp.zeros(idxT.shape, jnp.int32)
        for e in range(D):
            cnt = cnt + (idxT == idxT[e:e + 1, :]).astype(jnp.int32)
        m = jnp.max(cnt, axis=0, keepdims=True)
        return jnp.min(jnp.where(cnt == m, idxT, _INT_BIG), axis=0,
                       keepdims=True)

    mode_min = mode_of(minT_ref[...])
    mode_max = mode_of(maxT_ref[...])

    KP = cT0.shape[1]
    krange = lax.broadcasted_iota(jnp.int32, (KP, N), 0)
    oh_min = (krange == mode_min).astype(jnp.float32)
    oh_max = (krange == mode_max).astype(jnp.float32)
    posT = jnp.dot(cT0, oh_min, preferred_element_type=jnp.float32)
    negT = jnp.dot(cT0, oh_max, preferred_element_type=jnp.float32)

    eps = jnp.float32(1e-6)

    def pdist(a, b):
        return jnp.sqrt(jnp.sum((a - b + eps) ** 2, axis=0, keepdims=True))

    d_ap = pdist(fT, posT)
    d_an = pdist(fT, negT)
    d_pn = pdist(posT, negT)
    d_neg = jnp.minimum(d_an, d_pn)
    loss = jnp.sum(jnp.maximum(d_ap - d_neg + 1.0, 0.0)) / jnp.float32(N)
    out_ref[0, 0] = loss


def _mode_and_loss(fT, cT0, minT, maxT):
    out = pl.pallas_call(
        _loss_body,
        out_specs=pl.BlockSpec(memory_space=pltpu.SMEM),
        out_shape=jax.ShapeDtypeStruct((1, 1), jnp.float32),
    )(fT, cT0, minT, maxT)
    return out[0, 0]


def kernel(input_features, centroids):
    N, D = input_features.shape
    K, _ = centroids.shape
    KP = 1024
    assert K <= KP

    fT = input_features.T
    cT = centroids.T
    pad_inf = jnp.full((D, KP - K), jnp.inf, jnp.float32)
    pad_zero = jnp.zeros((D, KP - K), jnp.float32)
    cT_inf = jnp.concatenate([cT, pad_inf], axis=1)
    cT0 = jnp.concatenate([cT, pad_zero], axis=1)

    svals, ridx, ext = _sort_columns(cT_inf, K)
    minT, maxT = _sc_search(fT, svals, ridx, ext, K)
    return _mode_and_loss(fT, cT0, minT, maxT)
